# baseline (speedup 1.0000x reference)
import functools

import jax
import jax.numpy as jnp
from jax import lax
from jax.experimental import pallas as pl
from jax.experimental.pallas import tpu as pltpu
from jax.experimental.pallas import tpu_sc as plsc

_HASH_BUCKETS = 1000000
_DIM = 32
_B, _L = 4096, 200
_NW = 32
_ROWS_PER_W = _B // _NW
_IDS_W = _ROWS_PER_W * _L
_HALF = 112
_REST = _L - _HALF
_DEPTH = 4

_PERM = [c for k in range(16) for c in (k, 16 + k)]


def _mod1m(x):
    q = (x.astype(jnp.float32) * jnp.float32(1e-6)).astype(jnp.int32)
    r = x - q.astype(jnp.uint32) * jnp.uint32(1000000)
    r = jnp.where(r >= jnp.uint32(0x80000000), r + jnp.uint32(1000000), r)
    r = jnp.where(r >= jnp.uint32(1000000), r - jnp.uint32(1000000), r)
    return r.astype(jnp.int32)


_SCRATCH = [pltpu.VMEM((8 + _IDS_W + 16,), jnp.int32)]
for _ in range(_DEPTH):
    _SCRATCH += [
        pltpu.VMEM((_HALF,), jnp.int32),
        pltpu.VMEM((_HALF,), jnp.int32),
        pltpu.VMEM((_HALF,), jnp.int32),
        pltpu.VMEM((_HALF,), jnp.int32),
        pltpu.VMEM((_L, _DIM), jnp.bfloat16),
        pltpu.VMEM((_L, _DIM), jnp.bfloat16),
        pltpu.VMEM((_L, _DIM), jnp.float32),
        pltpu.SemaphoreType.DMA,
        pltpu.SemaphoreType.DMA,
    ]
_SCRATCH.append(pltpu.SemaphoreType.DMA)


@functools.partial(
    pl.kernel,
    mesh=plsc.VectorSubcoreMesh(core_axis_name="c", subcore_axis_name="s"),
    out_type=jax.ShapeDtypeStruct((_B, _L, _DIM), jnp.float32),
    compiler_params=pltpu.CompilerParams(use_tc_tiling_on_sc=False,
                                         needs_layout_passes=False),
    scratch_types=_SCRATCH,
)
def _sc_embed(ids_hbm, tri_hbm, four_hbm, out_hbm, ids_big, *rest):
    sets = []
    for s in range(_DEPTH):
        sets.append(rest[9 * s:9 * s + 9])
    sem_ids = rest[9 * _DEPTH]

    wid = lax.axis_index("s") * 2 + lax.axis_index("c")
    wb = wid * jnp.int32(_IDS_W)
    wrow = wid * jnp.int32(_ROWS_PER_W)
    zeros16 = jnp.zeros((16,), jnp.int32)
    lane = lax.iota(jnp.int32, 16)

    ids_big[pl.ds(0, 16)] = zeros16
    ids_big[pl.ds(8 + _IDS_W, 16)] = zeros16

    idsc = pltpu.make_async_copy(
        ids_hbm.at[pl.ds(wb, _IDS_W)], ids_big.at[pl.ds(8, _IDS_W)], sem_ids)
    idsc.start()
    idsc.wait()

    def hash_row(rl, ita, itb, ifa, ifb):
        base0 = rl * jnp.int32(_L) + jnp.int32(8)
        for j in range(13):
            base = base0 + jnp.int32(16 * j)
            v3 = ids_big[pl.ds(base, 16)].astype(jnp.uint32)
            v2 = ids_big[pl.ds(base - 1, 16)].astype(jnp.uint32)
            v1 = ids_big[pl.ds(base - 2, 16)].astype(jnp.uint32)
            v0 = ids_big[pl.ds(base - 3, 16)].astype(jnp.uint32)
            if j == 0:
                v2 = jnp.where(lane >= 1, v2, jnp.uint32(0))
                v1 = jnp.where(lane >= 2, v1, jnp.uint32(0))
                v0 = jnp.where(lane >= 3, v0, jnp.uint32(0))
            tri = v1 + v2 * jnp.uint32(257) + v3 * jnp.uint32(65537)
            four = (v0 + v1 * jnp.uint32(257) + v2 * jnp.uint32(65537)
                    + v3 * jnp.uint32(9973))
            ti = _mod1m(tri)
            fi = _mod1m(four)
            if j < 7:
                ita[pl.ds(j * 16, 16)] = ti
                ifa[pl.ds(j * 16, 16)] = fi
            else:
                itb[pl.ds((j - 7) * 16, 16)] = ti
                ifb[pl.ds((j - 7) * 16, 16)] = fi

    def g_copies(st):
        ita, itb, ifa, ifb, rt, rf = st[0:6]
        gsem = st[7]
        return (
            pltpu.make_async_copy(tri_hbm.at[ita], rt.at[pl.ds(0, _HALF)], gsem),
            pltpu.make_async_copy(tri_hbm.at[itb.at[pl.ds(0, _REST)]],
                                  rt.at[pl.ds(_HALF, _REST)], gsem),
            pltpu.make_async_copy(four_hbm.at[ifa], rf.at[pl.ds(0, _HALF)], gsem),
            pltpu.make_async_copy(four_hbm.at[ifb.at[pl.ds(0, _REST)]],
                                  rf.at[pl.ds(_HALF, _REST)], gsem),
        )

    def o_copy(st, rl):
        return pltpu.make_async_copy(
            st[6].at[pl.ds(0, _L)], out_hbm.at[wrow + rl], st[8])

    def sum_convert_rows(st):
        rt, rf, ob = st[4], st[5], st[6]

        def body(p, carry):
            pos = p * jnp.int32(4)
            for q in range(4):
                pq = pos + jnp.int32(q)
                a = rt[pq, :] + rf[pq, :]
                lo, hi = plsc.unpack(a, format=plsc.PackFormat.INTERLEAVED)
                ob[pq, pl.ds(0, 16)] = lo
                ob[pq, pl.ds(16, 16)] = hi
            return carry

        lax.fori_loop(jnp.int32(0), jnp.int32(_L // 4), body, jnp.int32(0))

    for s in range(_DEPTH):
        st = sets[s]
        hash_row(jnp.int32(s), st[0], st[1], st[2], st[3])
        for c in g_copies(st):
            c.start()

    n_iter = _ROWS_PER_W // _DEPTH

    def body(i, carry):
        for s in range(_DEPTH):
            st = sets[s]
            rl = _DEPTH * i + jnp.int32(s)
            for c in g_copies(st):
                c.wait()
            sum_convert_rows(st)
            o_copy(st, rl).start()
        for s in range(_DEPTH):
            st = sets[s]
            rl_next = jnp.minimum(_DEPTH * i + jnp.int32(_DEPTH + s),
                                  jnp.int32(_ROWS_PER_W - 1))
            hash_row(rl_next, st[0], st[1], st[2], st[3])
            o_copy(st, rl_next).wait()
            @pl.when(i < jnp.int32(n_iter - 1))
            def _():
                for c in g_copies(st):
                    c.start()
        return carry

    lax.fori_loop(jnp.int32(0), jnp.int32(n_iter), body, jnp.int32(0))


def kernel(input_ids, trigram_w, fourgram_w):
    ids32 = input_ids.astype(jnp.int32).reshape(_B * _L)
    perm = jnp.array(_PERM, dtype=jnp.int32)
    tri_bf = trigram_w[:, perm].astype(jnp.bfloat16)
    four_bf = fourgram_w[:, perm].astype(jnp.bfloat16)
    return _sc_embed(ids32, tri_bf, four_bf)


# speedup vs baseline: 1.1580x; 1.1580x over previous
import functools

import jax
import jax.numpy as jnp
from jax import lax
from jax.experimental import pallas as pl
from jax.experimental.pallas import tpu as pltpu
from jax.experimental.pallas import tpu_sc as plsc

_HASH_BUCKETS = 1000000
_DIM = 32
_B, _L = 4096, 200
_NW = 32
_ROWS_PER_W = _B // _NW
_IDS_W = _ROWS_PER_W * _L
_HALF = 112
_REST = _L - _HALF
_DEPTH = 4

def _mod1m(x):
    q = (x.astype(jnp.float32) * jnp.float32(1e-6)).astype(jnp.int32)
    r = x - q.astype(jnp.uint32) * jnp.uint32(1000000)
    r = jnp.where(r >= jnp.uint32(0x80000000), r + jnp.uint32(1000000), r)
    r = jnp.where(r >= jnp.uint32(1000000), r - jnp.uint32(1000000), r)
    return r.astype(jnp.int32)


_SCRATCH = [pltpu.VMEM((8 + _IDS_W + 16,), jnp.int32)]
for _ in range(_DEPTH):
    _SCRATCH += [
        pltpu.VMEM((_HALF,), jnp.int32),
        pltpu.VMEM((_HALF,), jnp.int32),
        pltpu.VMEM((_HALF,), jnp.int32),
        pltpu.VMEM((_HALF,), jnp.int32),
        pltpu.VMEM((_L, _DIM), jnp.bfloat16),
        pltpu.VMEM((_L, _DIM), jnp.bfloat16),
        pltpu.SemaphoreType.DMA,
        pltpu.SemaphoreType.DMA,
    ]
_SCRATCH.append(pltpu.SemaphoreType.DMA)


@functools.partial(
    pl.kernel,
    mesh=plsc.VectorSubcoreMesh(core_axis_name="c", subcore_axis_name="s"),
    out_type=jax.ShapeDtypeStruct((_B, _L, _DIM), jnp.bfloat16),
    compiler_params=pltpu.CompilerParams(use_tc_tiling_on_sc=False),
    scratch_types=_SCRATCH,
)
def _sc_embed(ids_hbm, tri_hbm, four_hbm, out_hbm, ids_big, *rest):
    sets = []
    for s in range(_DEPTH):
        sets.append(rest[8 * s:8 * s + 8])
    sem_ids = rest[8 * _DEPTH]

    wid = lax.axis_index("s") * 2 + lax.axis_index("c")
    wb = wid * jnp.int32(_IDS_W)
    wrow = wid * jnp.int32(_ROWS_PER_W)
    zeros16 = jnp.zeros((16,), jnp.int32)
    lane = lax.iota(jnp.int32, 16)

    ids_big[pl.ds(0, 16)] = zeros16
    ids_big[pl.ds(8 + _IDS_W, 16)] = zeros16

    idsc = pltpu.make_async_copy(
        ids_hbm.at[pl.ds(wb, _IDS_W)], ids_big.at[pl.ds(8, _IDS_W)], sem_ids)
    idsc.start()
    idsc.wait()

    def hash_row(rl, ita, itb, ifa, ifb):
        base0 = rl * jnp.int32(_L) + jnp.int32(8)
        for j in range(13):
            base = base0 + jnp.int32(16 * j)
            v3 = ids_big[pl.ds(base, 16)].astype(jnp.uint32)
            v2 = ids_big[pl.ds(base - 1, 16)].astype(jnp.uint32)
            v1 = ids_big[pl.ds(base - 2, 16)].astype(jnp.uint32)
            v0 = ids_big[pl.ds(base - 3, 16)].astype(jnp.uint32)
            if j == 0:
                v2 = jnp.where(lane >= 1, v2, jnp.uint32(0))
                v1 = jnp.where(lane >= 2, v1, jnp.uint32(0))
                v0 = jnp.where(lane >= 3, v0, jnp.uint32(0))
            tri = v1 + v2 * jnp.uint32(257) + v3 * jnp.uint32(65537)
            four = (v0 + v1 * jnp.uint32(257) + v2 * jnp.uint32(65537)
                    + v3 * jnp.uint32(9973))
            ti = _mod1m(tri)
            fi = _mod1m(four)
            if j < 7:
                ita[pl.ds(j * 16, 16)] = ti
                ifa[pl.ds(j * 16, 16)] = fi
            else:
                itb[pl.ds((j - 7) * 16, 16)] = ti
                ifb[pl.ds((j - 7) * 16, 16)] = fi

    def g_copies(st):
        ita, itb, ifa, ifb, rt, rf = st[0:6]
        gsem = st[6]
        return (
            pltpu.make_async_copy(tri_hbm.at[ita], rt.at[pl.ds(0, _HALF)], gsem),
            pltpu.make_async_copy(tri_hbm.at[itb.at[pl.ds(0, _REST)]],
                                  rt.at[pl.ds(_HALF, _REST)], gsem),
            pltpu.make_async_copy(four_hbm.at[ifa], rf.at[pl.ds(0, _HALF)], gsem),
            pltpu.make_async_copy(four_hbm.at[ifb.at[pl.ds(0, _REST)]],
                                  rf.at[pl.ds(_HALF, _REST)], gsem),
        )

    def o_copy(st, rl):
        return pltpu.make_async_copy(
            st[4].at[pl.ds(0, _L)], out_hbm.at[wrow + rl], st[7])

    def sum_rows(st):
        rt, rf = st[4], st[5]

        def body(p, carry):
            pos = p * jnp.int32(4)
            for q in range(4):
                pq = pos + jnp.int32(q)
                rt[pq, :] = rt[pq, :] + rf[pq, :]
            return carry

        lax.fori_loop(jnp.int32(0), jnp.int32(_L // 4), body, jnp.int32(0))

    for s in range(_DEPTH):
        st = sets[s]
        hash_row(jnp.int32(s), st[0], st[1], st[2], st[3])
        for c in g_copies(st):
            c.start()

    n_iter = _ROWS_PER_W // _DEPTH

    def body(i, carry):
        for s in range(_DEPTH):
            st = sets[s]
            rl = _DEPTH * i + jnp.int32(s)
            for c in g_copies(st):
                c.wait()
            sum_rows(st)
            o_copy(st, rl).start()
        for s in range(_DEPTH):
            st = sets[s]
            rl_next = jnp.minimum(_DEPTH * i + jnp.int32(_DEPTH + s),
                                  jnp.int32(_ROWS_PER_W - 1))
            hash_row(rl_next, st[0], st[1], st[2], st[3])
            o_copy(st, rl_next).wait()
            @pl.when(i < jnp.int32(n_iter - 1))
            def _():
                for c in g_copies(st):
                    c.start()
        return carry

    lax.fori_loop(jnp.int32(0), jnp.int32(n_iter), body, jnp.int32(0))


def kernel(input_ids, trigram_w, fourgram_w):
    ids32 = input_ids.astype(jnp.int32).reshape(_B * _L)
    tri_bf = trigram_w.astype(jnp.bfloat16)
    four_bf = fourgram_w.astype(jnp.bfloat16)
    return _sc_embed(ids32, tri_bf, four_bf).astype(jnp.float32)


# speedup vs baseline: 1.4058x; 1.2140x over previous
import functools

import jax
import jax.numpy as jnp
from jax import lax
from jax.experimental import pallas as pl
from jax.experimental.pallas import tpu as pltpu
from jax.experimental.pallas import tpu_sc as plsc

_HASH_BUCKETS = 1000000
_DIM = 32
_B, _L = 4096, 200
_NW = 32
_ROWS_PER_W = _B // _NW
_IDS_W = _ROWS_PER_W * _L
_HALF = 112
_REST = _L - _HALF
_DEPTH = 4


def _mod1m(x):
    q = (x.astype(jnp.float32) * jnp.float32(1e-6)).astype(jnp.int32)
    r = x - q.astype(jnp.uint32) * jnp.uint32(1000000)
    r = jnp.where(r >= jnp.uint32(0x80000000), r + jnp.uint32(1000000), r)
    r = jnp.where(r >= jnp.uint32(1000000), r - jnp.uint32(1000000), r)
    return r.astype(jnp.int32)


_SCRATCH = [pltpu.VMEM((8 + _IDS_W + 16,), jnp.int32)]
for _ in range(_DEPTH):
    _SCRATCH += [
        pltpu.VMEM((_HALF,), jnp.int32),
        pltpu.VMEM((_HALF,), jnp.int32),
        pltpu.VMEM((_HALF,), jnp.int32),
        pltpu.VMEM((_HALF,), jnp.int32),
        pltpu.VMEM((_L, _DIM), jnp.float32),
        pltpu.VMEM((_L, _DIM), jnp.float32),
        pltpu.SemaphoreType.DMA,
        pltpu.SemaphoreType.DMA,
    ]
_SCRATCH.append(pltpu.SemaphoreType.DMA)


@functools.partial(
    pl.kernel,
    mesh=plsc.VectorSubcoreMesh(core_axis_name="c", subcore_axis_name="s"),
    out_type=jax.ShapeDtypeStruct((_B * _L, _DIM), jnp.float32),
    compiler_params=pltpu.CompilerParams(use_tc_tiling_on_sc=False),
    scratch_types=_SCRATCH,
)
def _sc_embed(ids_hbm, tri_hbm, four_hbm, out_hbm, ids_big, *rest):
    sets = []
    for s in range(_DEPTH):
        sets.append(rest[8 * s:8 * s + 8])
    sem_ids = rest[8 * _DEPTH]

    wid = lax.axis_index("s") * 2 + lax.axis_index("c")
    wb = wid * jnp.int32(_IDS_W)
    zeros16 = jnp.zeros((16,), jnp.int32)
    lane = lax.iota(jnp.int32, 16)

    ids_big[pl.ds(0, 16)] = zeros16
    ids_big[pl.ds(8 + _IDS_W, 16)] = zeros16
    for (ita, itb, ifa, ifb, rt, rf, gsem, osem) in sets:
        itb[pl.ds(96, 16)] = zeros16
        ifb[pl.ds(96, 16)] = zeros16

    idsc = pltpu.make_async_copy(
        ids_hbm.at[pl.ds(wb, _IDS_W)], ids_big.at[pl.ds(8, _IDS_W)], sem_ids)
    idsc.start()
    idsc.wait()

    def hash_row(rl, ita, itb, ifa, ifb):
        base0 = rl * jnp.int32(_L) + jnp.int32(8)
        for j in range(13):
            base = base0 + jnp.int32(16 * j)
            v3 = ids_big[pl.ds(base, 16)].astype(jnp.uint32)
            v2 = ids_big[pl.ds(base - 1, 16)].astype(jnp.uint32)
            v1 = ids_big[pl.ds(base - 2, 16)].astype(jnp.uint32)
            v0 = ids_big[pl.ds(base - 3, 16)].astype(jnp.uint32)
            if j == 0:
                v2 = jnp.where(lane >= 1, v2, jnp.uint32(0))
                v1 = jnp.where(lane >= 2, v1, jnp.uint32(0))
                v0 = jnp.where(lane >= 3, v0, jnp.uint32(0))
            tri = v1 + v2 * jnp.uint32(257) + v3 * jnp.uint32(65537)
            four = (v0 + v1 * jnp.uint32(257) + v2 * jnp.uint32(65537)
                    + v3 * jnp.uint32(9973))
            ti = _mod1m(tri)
            fi = _mod1m(four)
            if j < 7:
                ita[pl.ds(j * 16, 16)] = ti
                ifa[pl.ds(j * 16, 16)] = fi
            else:
                itb[pl.ds((j - 7) * 16, 16)] = ti
                ifb[pl.ds((j - 7) * 16, 16)] = fi

    def g_copies(st):
        ita, itb, ifa, ifb, rt, rf, gsem, osem = st
        return (
            pltpu.make_async_copy(tri_hbm.at[ita], rt.at[pl.ds(0, _HALF)], gsem),
            pltpu.make_async_copy(tri_hbm.at[itb.at[pl.ds(0, _REST)]],
                                  rt.at[pl.ds(_HALF, _REST)], gsem),
            pltpu.make_async_copy(four_hbm.at[ifa], rf.at[pl.ds(0, _HALF)], gsem),
            pltpu.make_async_copy(four_hbm.at[ifb.at[pl.ds(0, _REST)]],
                                  rf.at[pl.ds(_HALF, _REST)], gsem),
        )

    def o_copy(st, rl):
        base = wb + rl * jnp.int32(_L)
        return pltpu.make_async_copy(
            st[4].at[pl.ds(0, _L)], out_hbm.at[pl.ds(base, _L)], st[7])

    def add_rows(st):
        rt, rf = st[4], st[5]

        def add_body(p, carry):
            pos = p * jnp.int32(4)
            for q in range(4):
                pq = pos + jnp.int32(q)
                for h in (0, 16):
                    rt[pq, pl.ds(h, 16)] = rt[pq, pl.ds(h, 16)] + rf[pq, pl.ds(h, 16)]
            return carry

        lax.fori_loop(jnp.int32(0), jnp.int32(_L // 4), add_body, jnp.int32(0))

    for s in range(_DEPTH):
        st = sets[s]
        hash_row(jnp.int32(s), st[0], st[1], st[2], st[3])
        for c in g_copies(st):
            c.start()

    n_iter = _ROWS_PER_W // _DEPTH

    def body(i, carry):
        for s in range(_DEPTH):
            st = sets[s]
            rl = _DEPTH * i + jnp.int32(s)
            for c in g_copies(st):
                c.wait()
            add_rows(st)
            o_copy(st, rl).start()
        for s in range(_DEPTH):
            st = sets[s]
            rl_next = jnp.minimum(_DEPTH * i + jnp.int32(_DEPTH + s),
                                  jnp.int32(_ROWS_PER_W - 1))
            hash_row(rl_next, st[0], st[1], st[2], st[3])
            o_copy(st, rl_next).wait()
            @pl.when(i < jnp.int32(n_iter - 1))
            def _():
                for c in g_copies(st):
                    c.start()
        return carry

    lax.fori_loop(jnp.int32(0), jnp.int32(n_iter), body, jnp.int32(0))


def kernel(input_ids, trigram_w, fourgram_w):
    ids32 = input_ids.astype(jnp.int32).reshape(_B * _L)
    out = _sc_embed(ids32, trigram_w, fourgram_w)
    return out.reshape(_B, _L, _DIM)
